# baseline (speedup 1.0000x reference)
import functools

import jax
import jax.numpy as jnp
from jax import lax
from jax.experimental import pallas as pl
from jax.experimental.pallas import tpu as pltpu
from jax.experimental.pallas import tpu_sc as plsc

_NUM_CORES = 2
_NUM_SUBCORES = 16
_NUM_WORKERS = _NUM_CORES * _NUM_SUBCORES


@functools.lru_cache(maxsize=None)
def _make_gather_kernel(num_rows: int, dim: int, chunk: int, nbuf: int):
    rows_per_worker = num_rows // _NUM_WORKERS
    num_chunks = rows_per_worker // chunk
    look = nbuf // 2
    assert num_rows % _NUM_WORKERS == 0
    assert rows_per_worker % chunk == 0
    assert num_chunks % nbuf == 0 and num_chunks >= 2 * nbuf
    assert chunk % 8 == 0

    mesh = plsc.VectorSubcoreMesh(
        core_axis_name="c",
        subcore_axis_name="s",
        num_cores=_NUM_CORES,
        num_subcores=_NUM_SUBCORES,
    )

    @functools.partial(
        pl.kernel,
        mesh=mesh,
        out_type=jax.ShapeDtypeStruct((num_rows, dim), jnp.float32),
        scratch_types=[
            pltpu.VMEM((rows_per_worker,), jnp.int32),
            [pltpu.VMEM((chunk, dim), jnp.float32) for _ in range(nbuf)],
            [pltpu.SemaphoreType.DMA for _ in range(nbuf)],
            [pltpu.SemaphoreType.DMA for _ in range(nbuf)],
        ],
    )
    def gather_kernel(table_hbm, idx_hbm, out_hbm, idx_v, bufs, gsems, ssems):
        wid = lax.axis_index("c") * _NUM_SUBCORES + lax.axis_index("s")
        base = wid * rows_per_worker
        pltpu.sync_copy(idx_hbm.at[pl.ds(base, rows_per_worker)], idx_v)

        def start_gather(chunk_id, b):
            off = chunk_id * chunk
            pltpu.async_copy(
                table_hbm.at[idx_v.at[pl.ds(off, chunk)]], bufs[b], gsems[b]
            )

        def wait_gather(b):
            pltpu.make_async_copy(
                table_hbm.at[pl.ds(0, chunk)], bufs[b], gsems[b]
            ).wait()

        def start_store(chunk_id, b):
            pltpu.async_copy(
                bufs[b], out_hbm.at[pl.ds(base + chunk_id * chunk, chunk)], ssems[b]
            )

        def wait_store(b):
            pltpu.make_async_copy(
                bufs[b], out_hbm.at[pl.ds(base, chunk)], ssems[b]
            ).wait()

        for i in range(look):
            start_gather(i, i)
        for i in range(look, nbuf):
            wait_gather(i - look)
            start_store(i - look, i - look)
            start_gather(i, i)

        def body(g, carry):
            for b in range(nbuf):
                i = g * nbuf + b
                wait_store(b)
                start_gather(i, b)
                b2 = (b + nbuf - look) % nbuf
                wait_gather(b2)
                start_store(i - look, b2)
            return carry

        lax.fori_loop(1, num_chunks // nbuf, body, 0, unroll=False)

        n = num_chunks
        for j in range(n - look, n):
            wait_gather(j % nbuf)
            start_store(j, j % nbuf)
        for b in range(nbuf):
            wait_store(b)

    return gather_kernel


def kernel(token_ids, weight):
    dim = weight.shape[1]
    b, s = token_ids.shape
    idx = token_ids.T.reshape(-1).astype(jnp.int32)
    gather = _make_gather_kernel(idx.shape[0], dim, 80, 8)
    out = gather(weight, idx)
    return out.reshape(s, b, dim).transpose(1, 0, 2)


# speedup vs baseline: 1.0228x; 1.0228x over previous
import functools

import jax
import jax.numpy as jnp
from jax import lax
from jax.experimental import pallas as pl
from jax.experimental.pallas import tpu as pltpu
from jax.experimental.pallas import tpu_sc as plsc

_NUM_CORES = 2
_NUM_SUBCORES = 16
_NUM_WORKERS = _NUM_CORES * _NUM_SUBCORES


@functools.lru_cache(maxsize=None)
def _make_gather_kernel(seq: int, batch: int, dim: int, nbuf: int):
    cols = batch // _NUM_WORKERS
    num_chunks = seq
    look = nbuf // 2
    assert batch % _NUM_WORKERS == 0
    assert cols % 8 == 0 and cols <= 128
    assert num_chunks % nbuf == 0 and num_chunks >= 2 * nbuf

    mesh = plsc.VectorSubcoreMesh(
        core_axis_name="c",
        subcore_axis_name="s",
        num_cores=_NUM_CORES,
        num_subcores=_NUM_SUBCORES,
    )

    @functools.partial(
        pl.kernel,
        mesh=mesh,
        out_type=jax.ShapeDtypeStruct((seq * batch, dim), jnp.float32),
        scratch_types=[
            pltpu.VMEM((seq, cols), jnp.int32),
            [pltpu.VMEM((cols, dim), jnp.float32) for _ in range(nbuf)],
            [pltpu.SemaphoreType.DMA for _ in range(nbuf)],
            [pltpu.SemaphoreType.DMA for _ in range(nbuf)],
        ],
    )
    def gather_kernel(table_hbm, idx_hbm, out_hbm, idx_v, bufs, gsems, ssems):
        wid = lax.axis_index("c") * _NUM_SUBCORES + lax.axis_index("s")
        col0 = wid * cols
        pltpu.sync_copy(idx_hbm.at[:, pl.ds(col0, cols)], idx_v)

        def start_gather(chunk_id, b):
            pltpu.async_copy(
                table_hbm.at[idx_v.at[chunk_id]], bufs[b], gsems[b]
            )

        def wait_gather(b):
            pltpu.make_async_copy(
                table_hbm.at[pl.ds(0, cols)], bufs[b], gsems[b]
            ).wait()

        def start_store(chunk_id, b):
            pltpu.async_copy(
                bufs[b], out_hbm.at[pl.ds(chunk_id * batch + col0, cols)], ssems[b]
            )

        def wait_store(b):
            pltpu.make_async_copy(
                bufs[b], out_hbm.at[pl.ds(col0, cols)], ssems[b]
            ).wait()

        for i in range(look):
            start_gather(i, i)
        for i in range(look, nbuf):
            wait_gather(i - look)
            start_store(i - look, i - look)
            start_gather(i, i)

        def body(g, carry):
            for b in range(nbuf):
                i = g * nbuf + b
                wait_store(b)
                start_gather(i, b)
                b2 = (b + nbuf - look) % nbuf
                wait_gather(b2)
                start_store(i - look, b2)
            return carry

        lax.fori_loop(1, num_chunks // nbuf, body, 0, unroll=False)

        n = num_chunks
        for j in range(n - look, n):
            wait_gather(j % nbuf)
            start_store(j, j % nbuf)
        for b in range(nbuf):
            wait_store(b)

    return gather_kernel


def kernel(token_ids, weight):
    dim = weight.shape[1]
    b, s = token_ids.shape
    ids = token_ids.T.astype(jnp.int32)
    gather = _make_gather_kernel(s, b, dim, 5)
    out = gather(weight, ids)
    return out.reshape(s, b, dim).transpose(1, 0, 2)
